# baseline (speedup 1.0000x reference)
import jax
import jax.numpy as jnp
from jax.experimental import pallas as pl
from jax.experimental.pallas import tpu as pltpu

N_CTX = 2048
N_EMBD = 768
N_EXP = 64
B = 4
FLAT = N_CTX * N_EMBD

CHUNK = 32768
N_STEPS = FLAT // CHUNK
N_CORES = 2
STEPS_PER_CORE = N_STEPS // N_CORES


def _matmul_kernel(x_ref, w_ref, o_ref, acc_ref):
    j = pl.program_id(1)
    part = jax.lax.dot_general(
        x_ref[...], w_ref[...],
        dimension_numbers=(((1,), (1,)), ((), ())),
        preferred_element_type=jnp.float32,
    )

    @pl.when(j == 0)
    def _init():
        acc_ref[...] = part

    @pl.when(j > 0)
    def _acc():
        acc_ref[...] = acc_ref[...] + part

    @pl.when(j == STEPS_PER_CORE - 1)
    def _flush():
        o_ref[0] = acc_ref[...]


def _route_kernel(y_ref, b_ref, o_ref):
    y = y_ref[0] + y_ref[1] + b_ref[...]
    col = jax.lax.broadcasted_iota(jnp.int32, (B, N_EXP), 1)
    v1 = jnp.max(y, axis=1, keepdims=True)
    i1 = jnp.min(jnp.where(y == v1, col, N_EXP), axis=1, keepdims=True)
    sel1 = col == i1
    y2 = jnp.where(sel1, -jnp.inf, y)
    v2 = jnp.max(y2, axis=1, keepdims=True)
    i2 = jnp.min(jnp.where(y2 == v2, col, N_EXP), axis=1, keepdims=True)
    sel2 = col == i2
    mask = jnp.where(sel1 | sel2, y, 0.0)
    m = jnp.max(mask, axis=1, keepdims=True)
    e = jnp.exp(mask - m)
    o_ref[...] = e / jnp.sum(e, axis=1, keepdims=True)


@jax.jit
def kernel(x, W, b):
    xf = x.reshape(B, FLAT)
    b2 = b.reshape(1, N_EXP)
    partials = pl.pallas_call(
        _matmul_kernel,
        grid=(N_CORES, STEPS_PER_CORE),
        in_specs=[
            pl.BlockSpec((B, CHUNK), lambda c, j: (0, c * STEPS_PER_CORE + j)),
            pl.BlockSpec((N_EXP, CHUNK), lambda c, j: (0, c * STEPS_PER_CORE + j)),
        ],
        out_specs=pl.BlockSpec((1, B, N_EXP), lambda c, j: (c, 0, 0)),
        out_shape=jax.ShapeDtypeStruct((N_CORES, B, N_EXP), jnp.float32),
        scratch_shapes=[pltpu.VMEM((B, N_EXP), jnp.float32)],
        compiler_params=pltpu.CompilerParams(
            dimension_semantics=("parallel", "arbitrary"),
        ),
    )(xf, W)
    return pl.pallas_call(
        _route_kernel,
        in_specs=[
            pl.BlockSpec((N_CORES, B, N_EXP), lambda: (0, 0, 0)),
            pl.BlockSpec((1, N_EXP), lambda: (0, 0)),
        ],
        out_specs=pl.BlockSpec((B, N_EXP), lambda: (0, 0)),
        out_shape=jax.ShapeDtypeStruct((B, N_EXP), jnp.float32),
    )(partials, b2)
